# baseline (speedup 1.0000x reference)
import functools

import jax
import jax.numpy as jnp
from jax import lax
from jax.experimental import pallas as pl
from jax.experimental.pallas import tpu as pltpu
from jax.experimental.pallas import tpu_sc as plsc

_LANES = 16


@functools.lru_cache(maxsize=None)
def _make_sorted_gather(num_rows, dim, batch):
    info = plsc.get_sparse_core_info()
    nc, ns = info.num_cores, info.num_subcores
    nw = nc * ns
    assert dim == 64 and batch % (nw * 128) == 0
    b_per_w = batch // nw
    tiles_j = -(-num_rows // 128)
    bw = 256
    clamp = tiles_j * 128 - bw

    mesh = plsc.VectorSubcoreMesh(core_axis_name="c", subcore_axis_name="s")

    @functools.partial(
        pl.kernel,
        mesh=mesh,
        out_type=jax.ShapeDtypeStruct((batch * dim,), jnp.float32),
        scratch_types=[
            pltpu.SMEM((b_per_w,), jnp.int32),
            pltpu.SMEM((b_per_w,), jnp.int32),
            pltpu.SMEM((b_per_w + 1,), jnp.int32),
            pltpu.VMEM((b_per_w,), jnp.int32),
            pltpu.VMEM((4, dim, bw), jnp.float32),
            pltpu.VMEM((b_per_w * dim,), jnp.float32),
        ] + [pltpu.SemaphoreType.DMA] * 5,
        compiler_params=pltpu.CompilerParams(needs_layout_passes=False),
    )
    def gather_kernel(table_hbm, sidx_hbm, out_hbm, idx_s, dj_s, dm_s, idx_v,
                      blk_v, stg_v, *sems_all):
        sems = sems_all[:4]
        aux_sem = sems_all[4]
        wid = lax.axis_index("s") * nc + lax.axis_index("c")
        base = wid * b_per_w
        pltpu.async_copy(sidx_hbm.at[pl.ds(base, b_per_w)], idx_v, aux_sem).wait()

        @pl.loop(0, b_per_w // _LANES)
        def _stage(mc):
            v = idx_v[pl.ds(mc * _LANES, _LANES)]
            for l in range(_LANES):
                idx_s[mc * _LANES + l] = v[l]

        def scan_body(m, carry):
            d, cur_j = carry
            j = lax.shift_right_logical(idx_s[m], 8)
            new = j != cur_j

            @pl.when(new)
            def _():
                dj_s[d] = j
                dm_s[d] = m

            return (d + jnp.where(new, 1, 0), j)

        nd, _ = pl.loop(0, b_per_w, init_carry=(jnp.int32(0), jnp.int32(-1)))(
            scan_body
        )
        dm_s[nd] = b_per_w

        def issue(d, sem):
            @pl.when(d < nd)
            def _():
                joff = jnp.minimum(dj_s[d] * bw, clamp)
                pltpu.async_copy(
                    table_hbm.at[:, pl.ds(joff, bw)],
                    blk_v.at[lax.rem(d, 4)],
                    sem,
                )

        for r in range(4):
            issue(jnp.int32(r), sems[r])

        rows0 = lax.iota(jnp.int32, _LANES)

        @pl.loop(0, nd, step=4)
        def _blocks(d0):
            for r in range(4):
                d = d0 + r

                @pl.when(d < nd)
                def _():
                    slot = lax.rem(d, 4)
                    pltpu.make_async_copy(
                        table_hbm.at[:, pl.ds(0, bw)], blk_v.at[slot], sems[r]
                    ).wait()
                    joff = jnp.minimum(dj_s[d] * bw, clamp)

                    def extract(m):
                        k = idx_s[m] - joff
                        cols = jnp.zeros((_LANES,), jnp.int32) + k
                        for q in range(dim // _LANES):
                            stg_v[pl.ds(m * dim + q * _LANES, _LANES)] = (
                                plsc.load_gather(
                                    blk_v.at[slot],
                                    [rows0 + q * _LANES, cols],
                                )
                            )

                    pl.loop(dm_s[d], dm_s[d + 1])(extract)
                    issue(d + 4, sems[r])

        pltpu.sync_copy(stg_v, out_hbm.at[pl.ds(base * dim, b_per_w * dim)])

    return gather_kernel


@functools.lru_cache(maxsize=None)
def _make_unpermute(dim, batch):
    info = plsc.get_sparse_core_info()
    nc, ns = info.num_cores, info.num_subcores
    nw = nc * ns
    b_per_w = batch // nw
    n_cb = b_per_w // 128

    mesh = plsc.VectorSubcoreMesh(core_axis_name="c", subcore_axis_name="s")

    @functools.partial(
        pl.kernel,
        mesh=mesh,
        out_type=jax.ShapeDtypeStruct((batch, dim), jnp.float32),
        scratch_types=[
            pltpu.VMEM((n_cb, 128), jnp.int32),
            pltpu.VMEM((b_per_w, dim), jnp.float32),
            pltpu.SemaphoreType.DMA,
            pltpu.SemaphoreType.DMA,
        ],
        compiler_params=pltpu.CompilerParams(use_tc_tiling_on_sc=False),
    )
    def scatter_kernel(rows_hbm, perm_hbm, out_hbm, perm_v, rows_v, sem,
                       aux_sem):
        wid = lax.axis_index("s") * nc + lax.axis_index("c")
        base = wid * b_per_w
        cp1 = pltpu.async_copy(
            perm_hbm.at[pl.ds(wid * n_cb, n_cb)], perm_v, aux_sem
        )
        cp2 = pltpu.async_copy(rows_hbm.at[pl.ds(base, b_per_w)], rows_v, sem)
        cp1.wait()
        cp2.wait()
        copies = []
        for q in range(n_cb):
            copies.append(
                pltpu.async_copy(
                    rows_v.at[pl.ds(q * 128, 128)],
                    out_hbm.at[perm_v.at[q]],
                    sem,
                )
            )
        for c in copies:
            c.wait()

    return scatter_kernel


def kernel(indices, table):
    batch, = indices.shape
    num_rows, dim = table.shape
    idx32 = indices.astype(jnp.int32)
    pos = lax.iota(jnp.int32, batch)
    sidx, perm = lax.sort([idx32, pos], dimension=0, num_keys=1)
    flat = _make_sorted_gather(num_rows, dim, batch)(
        jnp.swapaxes(table, 0, 1), sidx
    )
    rows_sorted = flat.reshape(batch, dim)
    return _make_unpermute(dim, batch)(rows_sorted, perm.reshape(-1, 128))


# speedup vs baseline: 1.1627x; 1.1627x over previous
import functools

import jax
import jax.numpy as jnp
from jax import lax
from jax.experimental import pallas as pl
from jax.experimental.pallas import tpu as pltpu
from jax.experimental.pallas import tpu_sc as plsc

_LANES = 16


@functools.lru_cache(maxsize=None)
def _make_sorted_gather(num_rows, dim, batch):
    info = plsc.get_sparse_core_info()
    nc, ns = info.num_cores, info.num_subcores
    nw = nc * ns
    assert dim == 64 and batch % (nw * 128) == 0
    b_per_w = batch // nw

    mesh = plsc.VectorSubcoreMesh(core_axis_name="c", subcore_axis_name="s")

    @functools.partial(
        pl.kernel,
        mesh=mesh,
        out_type=jax.ShapeDtypeStruct((batch * dim,), jnp.float32),
        scratch_types=[
            pltpu.SMEM((b_per_w,), jnp.int32),
            pltpu.SMEM((b_per_w,), jnp.int32),
            pltpu.SMEM((b_per_w + 1,), jnp.int32),
            pltpu.VMEM((b_per_w,), jnp.int32),
            pltpu.VMEM((8, dim, 128), jnp.float32),
            pltpu.VMEM((b_per_w * dim,), jnp.float32),
        ] + [pltpu.SemaphoreType.DMA] * 9,
        compiler_params=pltpu.CompilerParams(needs_layout_passes=False),
    )
    def gather_kernel(table_hbm, sidx_hbm, out_hbm, idx_s, dj_s, dm_s, idx_v,
                      blk_v, stg_v, *sems_all):
        sems = sems_all[:8]
        aux_sem = sems_all[8]
        wid = lax.axis_index("s") * nc + lax.axis_index("c")
        base = wid * b_per_w
        pltpu.async_copy(sidx_hbm.at[pl.ds(base, b_per_w)], idx_v, aux_sem).wait()

        @pl.loop(0, b_per_w // _LANES)
        def _stage(mc):
            v = idx_v[pl.ds(mc * _LANES, _LANES)]
            for l in range(_LANES):
                idx_s[mc * _LANES + l] = v[l]

        def scan_body(m, carry):
            d, cur_j = carry
            j = lax.shift_right_logical(idx_s[m], 7)
            new = j != cur_j

            @pl.when(new)
            def _():
                dj_s[d] = j
                dm_s[d] = m

            return (d + jnp.where(new, 1, 0), j)

        nd, _ = pl.loop(0, b_per_w, init_carry=(jnp.int32(0), jnp.int32(-1)))(
            scan_body
        )
        dm_s[nd] = b_per_w

        def issue(d, sem):
            @pl.when(d < nd)
            def _():
                j = dj_s[d]
                pltpu.async_copy(
                    table_hbm.at[:, pl.ds(j * 128, 128)],
                    blk_v.at[lax.rem(d, 8)],
                    sem,
                )

        for r in range(8):
            issue(jnp.int32(r), sems[r])

        rows0 = lax.iota(jnp.int32, _LANES)

        @pl.loop(0, nd, step=8)
        def _blocks(d0):
            for r in range(8):
                d = d0 + r

                @pl.when(d < nd)
                def _():
                    slot = lax.rem(d, 8)
                    pltpu.make_async_copy(
                        table_hbm.at[:, pl.ds(0, 128)], blk_v.at[slot], sems[r]
                    ).wait()

                    def extract(m):
                        k = lax.bitwise_and(idx_s[m], 127)
                        cols = jnp.zeros((_LANES,), jnp.int32) + k
                        for q in range(dim // _LANES):
                            stg_v[pl.ds(m * dim + q * _LANES, _LANES)] = (
                                plsc.load_gather(
                                    blk_v.at[slot],
                                    [rows0 + q * _LANES, cols],
                                )
                            )

                    pl.loop(dm_s[d], dm_s[d + 1])(extract)
                    issue(d + 8, sems[r])

        pltpu.sync_copy(stg_v, out_hbm.at[pl.ds(base * dim, b_per_w * dim)])

    return gather_kernel


@functools.lru_cache(maxsize=None)
def _make_unpermute(dim, batch):
    info = plsc.get_sparse_core_info()
    nc, ns = info.num_cores, info.num_subcores
    nw = nc * ns
    b_per_w = batch // nw
    n_cb = b_per_w // 128

    mesh = plsc.VectorSubcoreMesh(core_axis_name="c", subcore_axis_name="s")

    @functools.partial(
        pl.kernel,
        mesh=mesh,
        out_type=jax.ShapeDtypeStruct((batch, dim), jnp.float32),
        scratch_types=[
            pltpu.VMEM((n_cb, 128), jnp.int32),
            pltpu.VMEM((b_per_w, dim), jnp.float32),
            pltpu.SemaphoreType.DMA,
            pltpu.SemaphoreType.DMA,
        ],
        compiler_params=pltpu.CompilerParams(use_tc_tiling_on_sc=False),
    )
    def scatter_kernel(rows_hbm, perm_hbm, out_hbm, perm_v, rows_v, sem,
                       aux_sem):
        wid = lax.axis_index("s") * nc + lax.axis_index("c")
        base = wid * b_per_w
        cp1 = pltpu.async_copy(
            perm_hbm.at[pl.ds(wid * n_cb, n_cb)], perm_v, aux_sem
        )
        cp2 = pltpu.async_copy(rows_hbm.at[pl.ds(base, b_per_w)], rows_v, sem)
        cp1.wait()
        cp2.wait()
        copies = []
        for q in range(n_cb):
            copies.append(
                pltpu.async_copy(
                    rows_v.at[pl.ds(q * 128, 128)],
                    out_hbm.at[perm_v.at[q]],
                    sem,
                )
            )
        for c in copies:
            c.wait()

    return scatter_kernel


def kernel(indices, table):
    batch, = indices.shape
    num_rows, dim = table.shape
    idx32 = indices.astype(jnp.int32)
    pos = lax.iota(jnp.int32, batch)
    sidx, perm = lax.sort([idx32, pos], dimension=0, num_keys=1)
    flat = _make_sorted_gather(num_rows, dim, batch)(
        jnp.swapaxes(table, 0, 1), sidx
    )
    rows_sorted = flat.reshape(batch, dim)
    return _make_unpermute(dim, batch)(rows_sorted, perm.reshape(-1, 128))


# speedup vs baseline: 1.1633x; 1.0006x over previous
import functools

import jax
import jax.numpy as jnp
from jax import lax
from jax.experimental import pallas as pl
from jax.experimental.pallas import tpu as pltpu
from jax.experimental.pallas import tpu_sc as plsc

_LANES = 16


@functools.lru_cache(maxsize=None)
def _make_sorted_gather(num_rows, dim, batch):
    info = plsc.get_sparse_core_info()
    nc, ns = info.num_cores, info.num_subcores
    nw = nc * ns
    assert dim == 64 and batch % (nw * 128) == 0
    b_per_w = batch // nw

    mesh = plsc.VectorSubcoreMesh(core_axis_name="c", subcore_axis_name="s")

    @functools.partial(
        pl.kernel,
        mesh=mesh,
        out_type=jax.ShapeDtypeStruct((batch * dim,), jnp.float32),
        scratch_types=[
            pltpu.SMEM((b_per_w,), jnp.int32),
            pltpu.SMEM((b_per_w,), jnp.int32),
            pltpu.SMEM((b_per_w + 1,), jnp.int32),
            pltpu.VMEM((b_per_w,), jnp.int32),
            pltpu.VMEM((10, dim, 128), jnp.float32),
            pltpu.VMEM((b_per_w * dim,), jnp.float32),
        ] + [pltpu.SemaphoreType.DMA] * 11,
        compiler_params=pltpu.CompilerParams(needs_layout_passes=False),
    )
    def gather_kernel(table_hbm, sidx_hbm, out_hbm, idx_s, dj_s, dm_s, idx_v,
                      blk_v, stg_v, *sems_all):
        sems = sems_all[:10]
        aux_sem = sems_all[10]
        wid = lax.axis_index("s") * nc + lax.axis_index("c")
        base = wid * b_per_w
        pltpu.async_copy(sidx_hbm.at[pl.ds(base, b_per_w)], idx_v, aux_sem).wait()

        @pl.loop(0, b_per_w // _LANES)
        def _stage(mc):
            v = idx_v[pl.ds(mc * _LANES, _LANES)]
            for l in range(_LANES):
                idx_s[mc * _LANES + l] = v[l]

        def scan_body(m, carry):
            d, cur_j = carry
            j = lax.shift_right_logical(idx_s[m], 7)
            new = j != cur_j

            @pl.when(new)
            def _():
                dj_s[d] = j
                dm_s[d] = m

            return (d + jnp.where(new, 1, 0), j)

        nd, _ = pl.loop(0, b_per_w, init_carry=(jnp.int32(0), jnp.int32(-1)))(
            scan_body
        )
        dm_s[nd] = b_per_w

        def issue(d, sem):
            @pl.when(d < nd)
            def _():
                j = dj_s[d]
                pltpu.async_copy(
                    table_hbm.at[:, pl.ds(j * 128, 128)],
                    blk_v.at[lax.rem(d, 10)],
                    sem,
                )

        for r in range(10):
            issue(jnp.int32(r), sems[r])

        rows0 = lax.iota(jnp.int32, _LANES)

        @pl.loop(0, nd, step=10)
        def _blocks(d0):
            for r in range(10):
                d = d0 + r

                @pl.when(d < nd)
                def _():
                    slot = lax.rem(d, 10)
                    pltpu.make_async_copy(
                        table_hbm.at[:, pl.ds(0, 128)], blk_v.at[slot], sems[r]
                    ).wait()

                    def extract(m):
                        k = lax.bitwise_and(idx_s[m], 127)
                        cols = jnp.zeros((_LANES,), jnp.int32) + k
                        for q in range(dim // _LANES):
                            stg_v[pl.ds(m * dim + q * _LANES, _LANES)] = (
                                plsc.load_gather(
                                    blk_v.at[slot],
                                    [rows0 + q * _LANES, cols],
                                )
                            )

                    pl.loop(dm_s[d], dm_s[d + 1])(extract)
                    issue(d + 10, sems[r])

        pltpu.sync_copy(stg_v, out_hbm.at[pl.ds(base * dim, b_per_w * dim)])

    return gather_kernel


@functools.lru_cache(maxsize=None)
def _make_unpermute(dim, batch):
    info = plsc.get_sparse_core_info()
    nc, ns = info.num_cores, info.num_subcores
    nw = nc * ns
    b_per_w = batch // nw
    n_cb = b_per_w // 128

    mesh = plsc.VectorSubcoreMesh(core_axis_name="c", subcore_axis_name="s")

    @functools.partial(
        pl.kernel,
        mesh=mesh,
        out_type=jax.ShapeDtypeStruct((batch, dim), jnp.float32),
        scratch_types=[
            pltpu.VMEM((n_cb, 128), jnp.int32),
            pltpu.VMEM((b_per_w, dim), jnp.float32),
            pltpu.SemaphoreType.DMA,
            pltpu.SemaphoreType.DMA,
        ],
        compiler_params=pltpu.CompilerParams(use_tc_tiling_on_sc=False),
    )
    def scatter_kernel(rows_hbm, perm_hbm, out_hbm, perm_v, rows_v, sem,
                       aux_sem):
        wid = lax.axis_index("s") * nc + lax.axis_index("c")
        base = wid * b_per_w
        cp1 = pltpu.async_copy(
            perm_hbm.at[pl.ds(wid * n_cb, n_cb)], perm_v, aux_sem
        )
        cp2 = pltpu.async_copy(rows_hbm.at[pl.ds(base, b_per_w)], rows_v, sem)
        cp1.wait()
        cp2.wait()
        copies = []
        for q in range(n_cb):
            copies.append(
                pltpu.async_copy(
                    rows_v.at[pl.ds(q * 128, 128)],
                    out_hbm.at[perm_v.at[q]],
                    sem,
                )
            )
        for c in copies:
            c.wait()

    return scatter_kernel


def kernel(indices, table):
    batch, = indices.shape
    num_rows, dim = table.shape
    idx32 = indices.astype(jnp.int32)
    pos = lax.iota(jnp.int32, batch)
    sidx, perm = lax.sort([idx32, pos], dimension=0, num_keys=1)
    flat = _make_sorted_gather(num_rows, dim, batch)(
        jnp.swapaxes(table, 0, 1), sidx
    )
    rows_sorted = flat.reshape(batch, dim)
    return _make_unpermute(dim, batch)(rows_sorted, perm.reshape(-1, 128))
